# baseline (speedup 1.0000x reference)
import functools

import jax
import jax.numpy as jnp
from jax import lax
from jax.experimental import pallas as pl
from jax.experimental.pallas import tpu as pltpu

D_MODEL = 4096
N_EXP = 64
K = 8
T_BLK = 512
NSLOT = 4


def _router_body(x_hbm, wt_ref, idx_ref, gate_ref, logits_ref, x_vmem, sems):
    i = pl.program_id(0)
    n = pl.num_programs(0)

    def copy_in(blk, slot):
        pltpu.make_async_copy(
            x_hbm.at[pl.ds(blk * T_BLK, T_BLK), :],
            x_vmem.at[slot],
            sems.at[slot],
        ).start()

    @pl.when(i == 0)
    def _():
        for s in range(NSLOT - 1):
            copy_in(s, s)

    nxt = i + NSLOT - 1
    slot_nxt = lax.rem(nxt, NSLOT)

    @pl.when(nxt < n)
    def _():
        copy_in(nxt, slot_nxt)

    p = lax.rem(i, NSLOT)
    pltpu.make_async_copy(
        x_hbm.at[pl.ds(i * T_BLK, T_BLK), :], x_vmem.at[p], sems.at[p]
    ).wait()

    lt = jax.lax.dot_general(
        wt_ref[...],
        x_vmem[p],
        (((0,), (1,)), ((), ())),
        preferred_element_type=jnp.float32,
    )
    logits_ref[...] = lt.T

    iota = jax.lax.broadcasted_iota(jnp.int32, lt.shape, 0).astype(jnp.float32)
    cur = lt
    vals = []
    idxs = []
    for _ in range(K):
        m = jnp.max(cur, axis=0, keepdims=True)
        amax = jnp.min(
            jnp.where(cur == m, iota, jnp.float32(N_EXP)), axis=0, keepdims=True
        )
        vals.append(m)
        idxs.append(amax)
        cur = jnp.where(iota == amax, -jnp.inf, cur)

    tv = jnp.concatenate(vals, axis=0)
    ti = jnp.concatenate(idxs, axis=0)
    ev = jnp.exp(tv - tv[0:1, :])
    g = ev / jnp.sum(ev, axis=0, keepdims=True)
    gate_ref[...] = g.T
    idx_ref[...] = ti.T.astype(jnp.int32)


@jax.jit
def kernel(x, router_weights):
    b, s, d = x.shape
    n_tok = b * s
    x2 = x.reshape(n_tok, d)

    grid = (n_tok // T_BLK,)
    idx_out, gates, logits = pl.pallas_call(
        _router_body,
        grid=grid,
        in_specs=[
            pl.BlockSpec(memory_space=pl.MemorySpace.ANY),
            pl.BlockSpec((d, N_EXP), lambda i: (0, 0)),
        ],
        out_specs=[
            pl.BlockSpec((T_BLK, K), lambda i: (i, 0)),
            pl.BlockSpec((T_BLK, K), lambda i: (i, 0)),
            pl.BlockSpec((T_BLK, N_EXP), lambda i: (i, 0)),
        ],
        out_shape=[
            jax.ShapeDtypeStruct((n_tok, K), jnp.int32),
            jax.ShapeDtypeStruct((n_tok, K), jnp.float32),
            jax.ShapeDtypeStruct((n_tok, N_EXP), jnp.float32),
        ],
        scratch_shapes=[
            pltpu.VMEM((NSLOT, T_BLK, D_MODEL), jnp.float32),
            pltpu.SemaphoreType.DMA((NSLOT,)),
        ],
    )(x2, router_weights.T)

    return (
        idx_out.reshape(b, s, K),
        gates.reshape(b, s, K),
        logits.reshape(b, s, N_EXP),
    )


# speedup vs baseline: 1.0602x; 1.0602x over previous
import functools

import jax
import jax.numpy as jnp
from jax import lax
from jax.experimental import pallas as pl
from jax.experimental.pallas import tpu as pltpu

D_MODEL = 4096
N_EXP = 64
K = 8
T_BLK = 512
NSLOT = 4


def _router_body(x_hbm, wt_ref, idx_ref, gate_ref, logits_ref, x_vmem, sems):
    i = pl.program_id(0)
    n = pl.num_programs(0)

    def copy_in(blk, slot):
        h = T_BLK // 2
        pltpu.make_async_copy(
            x_hbm.at[pl.ds(blk * T_BLK, h), :],
            x_vmem.at[slot, pl.ds(0, h), :],
            sems.at[slot, 0],
        ).start()
        pltpu.make_async_copy(
            x_hbm.at[pl.ds(blk * T_BLK + h, h), :],
            x_vmem.at[slot, pl.ds(h, h), :],
            sems.at[slot, 1],
        ).start()

    @pl.when(i == 0)
    def _():
        for s in range(NSLOT - 1):
            copy_in(s, s)

    nxt = i + NSLOT - 1
    slot_nxt = lax.rem(nxt, NSLOT)

    @pl.when(nxt < n)
    def _():
        copy_in(nxt, slot_nxt)

    p = lax.rem(i, NSLOT)
    h = T_BLK // 2
    pltpu.make_async_copy(
        x_hbm.at[pl.ds(i * T_BLK, h), :], x_vmem.at[p, pl.ds(0, h), :], sems.at[p, 0]
    ).wait()
    pltpu.make_async_copy(
        x_hbm.at[pl.ds(i * T_BLK + h, h), :],
        x_vmem.at[p, pl.ds(h, h), :],
        sems.at[p, 1],
    ).wait()

    lt = jax.lax.dot_general(
        wt_ref[...],
        x_vmem[p],
        (((1,), (1,)), ((), ())),
        preferred_element_type=jnp.float32,
    )
    logits_ref[...] = lt.T

    iota = jax.lax.broadcasted_iota(jnp.int32, lt.shape, 0).astype(jnp.float32)
    cur = lt
    vals = []
    idxs = []
    for _ in range(K):
        m = jnp.max(cur, axis=0, keepdims=True)
        amax = jnp.min(
            jnp.where(cur == m, iota, jnp.float32(N_EXP)), axis=0, keepdims=True
        )
        vals.append(m)
        idxs.append(amax)
        cur = jnp.where(iota == amax, -jnp.inf, cur)

    tv = jnp.concatenate(vals, axis=0)
    ti = jnp.concatenate(idxs, axis=0)
    ev = jnp.exp(tv - tv[0:1, :])
    g = ev / jnp.sum(ev, axis=0, keepdims=True)
    gate_ref[...] = g.T
    idx_ref[...] = ti.T.astype(jnp.int32)


@jax.jit
def kernel(x, router_weights):
    b, s, d = x.shape
    n_tok = b * s
    x2 = x.reshape(n_tok, d)

    grid = (n_tok // T_BLK,)
    idx_out, gates, logits = pl.pallas_call(
        _router_body,
        grid=grid,
        in_specs=[
            pl.BlockSpec(memory_space=pl.MemorySpace.ANY),
            pl.BlockSpec((N_EXP, d), lambda i: (0, 0)),
        ],
        out_specs=[
            pl.BlockSpec((T_BLK, K), lambda i: (i, 0)),
            pl.BlockSpec((T_BLK, K), lambda i: (i, 0)),
            pl.BlockSpec((T_BLK, N_EXP), lambda i: (i, 0)),
        ],
        out_shape=[
            jax.ShapeDtypeStruct((n_tok, K), jnp.int32),
            jax.ShapeDtypeStruct((n_tok, K), jnp.float32),
            jax.ShapeDtypeStruct((n_tok, N_EXP), jnp.float32),
        ],
        scratch_shapes=[
            pltpu.VMEM((NSLOT, T_BLK, D_MODEL), jnp.float32),
            pltpu.SemaphoreType.DMA((NSLOT, 2)),
        ],
    )(x2, router_weights)

    return (
        idx_out.reshape(b, s, K),
        gates.reshape(b, s, K),
        logits.reshape(b, s, N_EXP),
    )


# speedup vs baseline: 1.0653x; 1.0048x over previous
import functools

import jax
import jax.numpy as jnp
from jax.experimental import pallas as pl

D_MODEL = 4096
N_EXP = 64
K = 8
T_BLK = 1024


def _router_body(x_ref, wt_ref, idx_ref, gate_ref, logits_ref):
    lt = jax.lax.dot_general(
        wt_ref[...], x_ref[...], (((1,), (1,)), ((), ())),
        preferred_element_type=jnp.float32)
    logits_ref[...] = lt.T
    iota = jax.lax.broadcasted_iota(jnp.int32, lt.shape, 0).astype(jnp.float32)
    cur = lt
    vals = []
    idxs = []
    for _ in range(K):
        m = jnp.max(cur, axis=0, keepdims=True)
        amax = jnp.min(
            jnp.where(cur == m, iota, jnp.float32(N_EXP)), axis=0, keepdims=True
        )
        vals.append(m)
        idxs.append(amax)
        cur = jnp.where(iota == amax, -jnp.inf, cur)

    tv = jnp.concatenate(vals, axis=0)
    ti = jnp.concatenate(idxs, axis=0)
    ev = jnp.exp(tv - tv[0:1, :])
    g = ev / jnp.sum(ev, axis=0, keepdims=True)
    gate_ref[...] = g.T
    idx_ref[...] = ti.T.astype(jnp.int32)


@jax.jit
def kernel(x, router_weights):
    b, s, d = x.shape
    n_tok = b * s
    x2 = x.reshape(n_tok, d)
    wt = router_weights

    grid = (n_tok // T_BLK,)
    idx_out, gates, logits = pl.pallas_call(
        _router_body,
        grid=grid,
        in_specs=[
            pl.BlockSpec((T_BLK, d), lambda i: (i, 0)),
            pl.BlockSpec((N_EXP, d), lambda i: (0, 0)),
        ],
        out_specs=[
            pl.BlockSpec((T_BLK, K), lambda i: (i, 0)),
            pl.BlockSpec((T_BLK, K), lambda i: (i, 0)),
            pl.BlockSpec((T_BLK, N_EXP), lambda i: (i, 0)),
        ],
        out_shape=[
            jax.ShapeDtypeStruct((n_tok, K), jnp.int32),
            jax.ShapeDtypeStruct((n_tok, K), jnp.float32),
            jax.ShapeDtypeStruct((n_tok, N_EXP), jnp.float32),
        ],
    )(x2, wt)

    return (
        idx_out.reshape(b, s, K),
        gates.reshape(b, s, K),
        logits.reshape(b, s, N_EXP),
    )
